# baseline (speedup 1.0000x reference)
import functools

import jax
import jax.numpy as jnp
from jax import lax
from jax.experimental import pallas as pl
from jax.experimental.pallas import tpu as pltpu
from jax.experimental.pallas import tpu_sc as plsc

B = 16384
D = 32
NC, NS = 2, 16
NW = NC * NS
BPW = B // NW
N_ACTOR, N_COUNTRY, N_TYPE = 20, 4, 8
NUM_USERS = 1000000


def _sc_gather_body(user_hbm, movie_hbm, actor_hbm, country_hbm, type_hbm,
                    uflat_hbm, mt_hbm, at_hbm, ct_hbm, tt_hbm,
                    u_out, m_out, a_out, c_out, t_out,
                    uidx_v, midx_v, aidx_v, cidx_v, tidx_v,
                    eidx_v, du_v, acc_v, sem_u, sem):
    wid = lax.axis_index("s") * NC + lax.axis_index("c")
    base = wid * BPW

    pltpu.sync_copy(user_hbm.at[pl.ds(base, BPW)], uidx_v)
    pltpu.sync_copy(movie_hbm.at[pl.ds(base, BPW)], midx_v)
    pltpu.sync_copy(actor_hbm.at[:, pl.ds(base, BPW)], aidx_v)
    pltpu.sync_copy(country_hbm.at[:, pl.ds(base, BPW)], cidx_v)
    pltpu.sync_copy(type_hbm.at[:, pl.ds(base, BPW)], tidx_v)

    def gen_d(dd, carry):
        def gen_i(c, carry2):
            i16 = c * 16
            eidx_v[dd, pl.ds(i16, 16)] = uidx_v[pl.ds(i16, 16)] + dd * NUM_USERS
            return carry2
        lax.fori_loop(0, BPW // 16, gen_i, 0)
        return carry
    lax.fori_loop(0, D, gen_d, 0)

    def fire(dd, carry):
        pltpu.async_copy(uflat_hbm.at[eidx_v.at[dd]], du_v.at[dd], sem_u)
        return carry
    lax.fori_loop(0, D, fire, 0)

    pltpu.async_copy(mt_hbm.at[midx_v], acc_v, sem).wait()
    pltpu.sync_copy(acc_v, m_out.at[pl.ds(base, BPW)])

    def pooled(idx_v, n, table, out):
        pltpu.async_copy(table.at[idx_v.at[0]], acc_v, sem).wait()

        def body(j, carry):
            pltpu.async_copy(table.at[idx_v.at[j]], acc_v, sem, add=True).wait()
            return carry

        lax.fori_loop(1, n, body, 0)
        pltpu.sync_copy(acc_v, out.at[pl.ds(base, BPW)])

    pooled(aidx_v, N_ACTOR, at_hbm, a_out)
    pooled(cidx_v, N_COUNTRY, ct_hbm, c_out)
    pooled(tidx_v, N_TYPE, tt_hbm, t_out)

    def drain(dd, carry):
        pltpu.make_async_copy(uflat_hbm.at[eidx_v.at[0]], du_v.at[0], sem_u).wait()
        return carry
    lax.fori_loop(0, D, drain, 0)
    pltpu.sync_copy(du_v, u_out.at[:, pl.ds(base, BPW)])


def _sc_gather(user, movie, actor_t, country_t, type_t,
               uflat, movie_table, actor_table, country_table, type_table):
    emb = jax.ShapeDtypeStruct((B, D), jnp.float32)
    embT = jax.ShapeDtypeStruct((D, B), jnp.float32)
    run = pl.kernel(
        _sc_gather_body,
        out_type=(embT, emb, emb, emb, emb),
        mesh=plsc.VectorSubcoreMesh(core_axis_name="c", subcore_axis_name="s",
                                    num_cores=NC, num_subcores=NS),
        scratch_types=[
            pltpu.VMEM((BPW,), jnp.int32),
            pltpu.VMEM((BPW,), jnp.int32),
            pltpu.VMEM((N_ACTOR, BPW), jnp.int32),
            pltpu.VMEM((N_COUNTRY, BPW), jnp.int32),
            pltpu.VMEM((N_TYPE, BPW), jnp.int32),
            pltpu.VMEM((D, BPW), jnp.int32),
            pltpu.VMEM((D, BPW), jnp.float32),
            pltpu.VMEM((BPW, D), jnp.float32),
            pltpu.SemaphoreType.DMA,
            pltpu.SemaphoreType.DMA,
        ],
        compiler_params=pltpu.CompilerParams(use_tc_tiling_on_sc=False),
    )
    return run(user, movie, actor_t, country_t, type_t,
               uflat, movie_table, actor_table, country_table, type_table)


def _mlp_body(uT, m, a, c, t, w1, b1, w2, b2, w3, b3, out):
    f32 = jnp.float32
    dn0 = (((0,), (0,)), ((), ()))
    h = (lax.dot_general(uT[...], w1[0:D, :], dn0, preferred_element_type=f32)
         + jnp.dot(m[...], w1[D:2 * D, :], preferred_element_type=f32)
         + jnp.dot(a[...] * (1.0 / N_ACTOR), w1[2 * D:3 * D, :], preferred_element_type=f32)
         + jnp.dot(c[...] * (1.0 / N_COUNTRY), w1[3 * D:4 * D, :], preferred_element_type=f32)
         + jnp.dot(t[...] * (1.0 / N_TYPE), w1[4 * D:5 * D, :], preferred_element_type=f32)
         + b1[...])
    h = jnp.maximum(h, 0.0)
    h2 = jnp.maximum(jnp.dot(h, w2[...], preferred_element_type=f32) + b2[...], 0.0)
    out[...] = jnp.dot(h2, w3[...], preferred_element_type=f32) + b3[...]


def _mlp(uT, m, a, c, t, W1, b1, W2, b2, W3, b3):
    BM = 2048
    grid = (B // BM,)
    emb_spec = pl.BlockSpec((BM, D), lambda i: (i, 0))
    embT_spec = pl.BlockSpec((D, BM), lambda i: (0, i))
    full = lambda s: pl.BlockSpec(s, lambda i: tuple(0 for _ in s))
    return pl.pallas_call(
        _mlp_body,
        grid=grid,
        in_specs=[embT_spec, emb_spec, emb_spec, emb_spec, emb_spec,
                  full((5 * D, 64)), full((64,)), full((64, 32)), full((32,)),
                  full((32, 1)), full((1,))],
        out_specs=pl.BlockSpec((BM, 1), lambda i: (i, 0)),
        out_shape=jax.ShapeDtypeStruct((B, 1), jnp.float32),
    )(uT, m, a, c, t, W1, b1, W2, b2, W3, b3)


def kernel(user, movie, actor, country, movie_type,
           user_table, movie_table, actor_table, country_table, type_table,
           W1, b1, W2, b2, W3, b3):
    user = user.astype(jnp.int32)
    actor_t = actor.T
    country_t = country.T
    type_t = movie_type.T
    uflat = user_table.T.reshape(-1)
    u, m, a, c, t = _sc_gather(user, movie, actor_t, country_t, type_t,
                               uflat, movie_table, actor_table,
                               country_table, type_table)
    y = _mlp(u, m, a, c, t, W1, b1, W2, b2, W3, b3)
    return jnp.squeeze(y, axis=-1)


# speedup vs baseline: 8.0556x; 8.0556x over previous
import functools

import jax
import jax.numpy as jnp
from jax import lax
from jax.experimental import pallas as pl
from jax.experimental.pallas import tpu as pltpu
from jax.experimental.pallas import tpu_sc as plsc

B = 16384
D = 32
NC, NS = 2, 16
NW = NC * NS
BPW = B // NW
N_ACTOR, N_COUNTRY, N_TYPE = 20, 4, 8
NUM_USERS = 1000000
UW = 1 << 20
UCW = 1 << 16


def _sc_gather_body(user_hbm, movie_hbm, actor_hbm, country_hbm, type_hbm,
                    uflat_hbm, mt_hbm, at_hbm, ct_hbm, tt_hbm,
                    u_out, m_out, a_out, c_out, t_out,
                    uidx_v, midx_v, aidx_v, cidx_v, tidx_v,
                    eidx_v, du_v, acc_v, sem_u, sem):
    wid = lax.axis_index("s") * NC + lax.axis_index("c")
    base = wid * BPW

    pltpu.sync_copy(user_hbm.at[pl.ds(base, BPW)], uidx_v)
    pltpu.sync_copy(movie_hbm.at[pl.ds(base, BPW)], midx_v)
    pltpu.sync_copy(actor_hbm.at[:, pl.ds(base, BPW)], aidx_v)
    pltpu.sync_copy(country_hbm.at[:, pl.ds(base, BPW)], cidx_v)
    pltpu.sync_copy(type_hbm.at[:, pl.ds(base, BPW)], tidx_v)

    def gen_d(dd, carry):
        def gen_i(c, carry2):
            i16 = c * 16
            v = uidx_v[pl.ds(i16, 16)]
            eidx_v[dd, pl.ds(i16, 16)] = (
                ((v >> 16) << 19) + (v & 0xFFFF)
                + ((dd >> 3) << 23) + ((dd & 7) << 16))
            return carry2
        lax.fori_loop(0, BPW // 16, gen_i, 0)
        return carry
    lax.fori_loop(0, D, gen_d, 0)

    def fire(dd, carry):
        pltpu.async_copy(uflat_hbm.at[eidx_v.at[dd]], du_v.at[dd], sem_u)
        return carry
    lax.fori_loop(0, D, fire, 0)

    pltpu.async_copy(mt_hbm.at[midx_v], acc_v, sem).wait()
    pltpu.sync_copy(acc_v, m_out.at[pl.ds(base, BPW)])

    def pooled(idx_v, n, table, out):
        pltpu.async_copy(table.at[idx_v.at[0]], acc_v, sem).wait()

        def body(j, carry):
            pltpu.async_copy(table.at[idx_v.at[j]], acc_v, sem, add=True).wait()
            return carry

        lax.fori_loop(1, n, body, 0)
        pltpu.sync_copy(acc_v, out.at[pl.ds(base, BPW)])

    pooled(aidx_v, N_ACTOR, at_hbm, a_out)
    pooled(cidx_v, N_COUNTRY, ct_hbm, c_out)
    pooled(tidx_v, N_TYPE, tt_hbm, t_out)

    def drain(dd, carry):
        pltpu.make_async_copy(uflat_hbm.at[eidx_v.at[0]], du_v.at[0], sem_u).wait()
        return carry
    lax.fori_loop(0, D, drain, 0)
    pltpu.sync_copy(du_v, u_out.at[:, pl.ds(base, BPW)])


def _sc_gather(user, movie, actor_t, country_t, type_t,
               uflat, movie_table, actor_table, country_table, type_table):
    emb = jax.ShapeDtypeStruct((B, D), jnp.float32)
    embT = jax.ShapeDtypeStruct((D, B), jnp.float32)
    run = pl.kernel(
        _sc_gather_body,
        out_type=(embT, emb, emb, emb, emb),
        mesh=plsc.VectorSubcoreMesh(core_axis_name="c", subcore_axis_name="s",
                                    num_cores=NC, num_subcores=NS),
        scratch_types=[
            pltpu.VMEM((BPW,), jnp.int32),
            pltpu.VMEM((BPW,), jnp.int32),
            pltpu.VMEM((N_ACTOR, BPW), jnp.int32),
            pltpu.VMEM((N_COUNTRY, BPW), jnp.int32),
            pltpu.VMEM((N_TYPE, BPW), jnp.int32),
            pltpu.VMEM((D, BPW), jnp.int32),
            pltpu.VMEM((D, BPW), jnp.float32),
            pltpu.VMEM((BPW, D), jnp.float32),
            pltpu.SemaphoreType.DMA,
            pltpu.SemaphoreType.DMA,
        ],
        compiler_params=pltpu.CompilerParams(use_tc_tiling_on_sc=False),
    )
    return run(user, movie, actor_t, country_t, type_t,
               uflat, movie_table, actor_table, country_table, type_table)


def _detile_body(inp, out):
    out[...] = inp[...].reshape(8 * UCW)


def _detile(uT):
    grid = (D // 8, UW // UCW)
    return pl.pallas_call(
        _detile_body,
        grid=grid,
        in_specs=[pl.BlockSpec((8, UCW), lambda d8, k: (d8, k))],
        out_specs=pl.BlockSpec((8 * UCW,), lambda d8, k: (d8 * (UW // UCW) + k,)),
        out_shape=jax.ShapeDtypeStruct((D * UW,), jnp.float32),
    )(uT)


def _mlp_body(uT, m, a, c, t, w1, b1, w2, b2, w3, b3, out):
    f32 = jnp.float32
    dn0 = (((0,), (0,)), ((), ()))
    h = (lax.dot_general(uT[...], w1[0:D, :], dn0, preferred_element_type=f32)
         + jnp.dot(m[...], w1[D:2 * D, :], preferred_element_type=f32)
         + jnp.dot(a[...] * (1.0 / N_ACTOR), w1[2 * D:3 * D, :], preferred_element_type=f32)
         + jnp.dot(c[...] * (1.0 / N_COUNTRY), w1[3 * D:4 * D, :], preferred_element_type=f32)
         + jnp.dot(t[...] * (1.0 / N_TYPE), w1[4 * D:5 * D, :], preferred_element_type=f32)
         + b1[...])
    h = jnp.maximum(h, 0.0)
    h2 = jnp.maximum(jnp.dot(h, w2[...], preferred_element_type=f32) + b2[...], 0.0)
    out[...] = jnp.dot(h2, w3[...], preferred_element_type=f32) + b3[...]


def _mlp(uT, m, a, c, t, W1, b1, W2, b2, W3, b3):
    BM = 2048
    grid = (B // BM,)
    emb_spec = pl.BlockSpec((BM, D), lambda i: (i, 0))
    embT_spec = pl.BlockSpec((D, BM), lambda i: (0, i))
    full = lambda s: pl.BlockSpec(s, lambda i: tuple(0 for _ in s))
    return pl.pallas_call(
        _mlp_body,
        grid=grid,
        in_specs=[embT_spec, emb_spec, emb_spec, emb_spec, emb_spec,
                  full((5 * D, 64)), full((64,)), full((64, 32)), full((32,)),
                  full((32, 1)), full((1,))],
        out_specs=pl.BlockSpec((BM, 1), lambda i: (i, 0)),
        out_shape=jax.ShapeDtypeStruct((B, 1), jnp.float32),
    )(uT, m, a, c, t, W1, b1, W2, b2, W3, b3)


def kernel(user, movie, actor, country, movie_type,
           user_table, movie_table, actor_table, country_table, type_table,
           W1, b1, W2, b2, W3, b3):
    user = user.astype(jnp.int32)
    actor_t = actor.T
    country_t = country.T
    type_t = movie_type.T
    uflat = _detile(user_table.T)
    u, m, a, c, t = _sc_gather(user, movie, actor_t, country_t, type_t,
                               uflat, movie_table, actor_table,
                               country_table, type_table)
    y = _mlp(u, m, a, c, t, W1, b1, W2, b2, W3, b3)
    return jnp.squeeze(y, axis=-1)


# speedup vs baseline: 8.7521x; 1.0865x over previous
import functools

import jax
import jax.numpy as jnp
from jax import lax
from jax.experimental import pallas as pl
from jax.experimental.pallas import tpu as pltpu
from jax.experimental.pallas import tpu_sc as plsc

B = 16384
D = 32
NC, NS = 2, 16
NW = NC * NS
BPW = B // NW
N_ACTOR, N_COUNTRY, N_TYPE = 20, 4, 8
UCW = 1 << 16
KB_U = 16
KB_M = 2


def _scalar_offsets(idx, kb, dd):
    return ((((dd >> 3) * kb + (idx >> 16)) << 19)
            + ((dd & 7) << 16) + (idx & 0xFFFF))


def _sc_gather_body(user_hbm, movie_hbm, actor_hbm, country_hbm, type_hbm,
                    uflat_hbm, mflat_hbm, at_hbm, ct_hbm, tt_hbm,
                    u_out, m_out, a_out, c_out, t_out,
                    uidx_v, midx_v, aidx_v, cidx_v, tidx_v,
                    eu_v, em_v, du_v, dm_v, acc_a, acc_c, acc_t,
                    sem_s, sem_a, sem_c, sem_t):
    wid = lax.axis_index("s") * NC + lax.axis_index("c")
    base = wid * BPW

    pltpu.sync_copy(user_hbm.at[pl.ds(base, BPW)], uidx_v)
    pltpu.sync_copy(movie_hbm.at[pl.ds(base, BPW)], midx_v)
    pltpu.sync_copy(actor_hbm.at[pl.ds(0, 10), pl.ds(base, BPW)], aidx_v)
    pltpu.sync_copy(country_hbm.at[:, pl.ds(base, BPW)], cidx_v)
    pltpu.sync_copy(type_hbm.at[:, pl.ds(base, BPW)], tidx_v)

    def gen_d(dd, carry):
        def gen_i(c, carry2):
            i16 = c * 16
            eu_v[dd, pl.ds(i16, 16)] = _scalar_offsets(
                uidx_v[pl.ds(i16, 16)], KB_U, dd)
            em_v[dd, pl.ds(i16, 16)] = _scalar_offsets(
                midx_v[pl.ds(i16, 16)], KB_M, dd)
            return carry2
        lax.fori_loop(0, BPW // 16, gen_i, 0)
        return carry
    lax.fori_loop(0, D, gen_d, 0)

    def fire(dd, carry):
        pltpu.async_copy(uflat_hbm.at[eu_v.at[dd]], du_v.at[dd], sem_s)
        pltpu.async_copy(mflat_hbm.at[em_v.at[dd]], dm_v.at[dd], sem_s)
        return carry
    lax.fori_loop(0, D, fire, 0)

    def fire_p(table, idx_v, j, acc, sem, add):
        pltpu.async_copy(table.at[idx_v.at[j]], acc, sem, add=add)

    def wait_p(table, idx_v, acc, sem):
        pltpu.make_async_copy(table.at[idx_v.at[0]], acc, sem).wait()

    fire_p(at_hbm, aidx_v, 0, acc_a, sem_a, False)
    fire_p(ct_hbm, cidx_v, 0, acc_c, sem_c, False)
    fire_p(tt_hbm, tidx_v, 0, acc_t, sem_t, False)

    def step3(j, carry):
        wait_p(at_hbm, aidx_v, acc_a, sem_a)
        fire_p(at_hbm, aidx_v, j, acc_a, sem_a, True)
        wait_p(ct_hbm, cidx_v, acc_c, sem_c)
        fire_p(ct_hbm, cidx_v, j, acc_c, sem_c, True)
        wait_p(tt_hbm, tidx_v, acc_t, sem_t)
        fire_p(tt_hbm, tidx_v, j, acc_t, sem_t, True)
        return carry
    lax.fori_loop(1, N_COUNTRY, step3, 0)

    def step2(j, carry):
        wait_p(at_hbm, aidx_v, acc_a, sem_a)
        fire_p(at_hbm, aidx_v, j, acc_a, sem_a, True)
        wait_p(tt_hbm, tidx_v, acc_t, sem_t)
        fire_p(tt_hbm, tidx_v, j, acc_t, sem_t, True)
        return carry
    lax.fori_loop(N_COUNTRY, N_TYPE, step2, 0)

    def step1(j, carry):
        wait_p(at_hbm, aidx_v, acc_a, sem_a)
        fire_p(at_hbm, aidx_v, j, acc_a, sem_a, True)
        return carry
    lax.fori_loop(N_TYPE, 10, step1, 0)

    wait_p(at_hbm, aidx_v, acc_a, sem_a)
    pltpu.sync_copy(actor_hbm.at[pl.ds(10, 10), pl.ds(base, BPW)], aidx_v)
    fire_p(at_hbm, aidx_v, 0, acc_a, sem_a, True)

    def step1b(j, carry):
        wait_p(at_hbm, aidx_v, acc_a, sem_a)
        fire_p(at_hbm, aidx_v, j, acc_a, sem_a, True)
        return carry
    lax.fori_loop(1, 10, step1b, 0)

    wait_p(ct_hbm, cidx_v, acc_c, sem_c)
    pltpu.sync_copy(acc_c, c_out.at[pl.ds(base, BPW)])
    wait_p(tt_hbm, tidx_v, acc_t, sem_t)
    pltpu.sync_copy(acc_t, t_out.at[pl.ds(base, BPW)])
    wait_p(at_hbm, aidx_v, acc_a, sem_a)
    pltpu.sync_copy(acc_a, a_out.at[pl.ds(base, BPW)])

    def drain(dd, carry):
        pltpu.make_async_copy(uflat_hbm.at[eu_v.at[0]], du_v.at[0], sem_s).wait()
        pltpu.make_async_copy(mflat_hbm.at[em_v.at[0]], dm_v.at[0], sem_s).wait()
        return carry
    lax.fori_loop(0, D, drain, 0)
    pltpu.sync_copy(du_v, u_out.at[:, pl.ds(base, BPW)])
    pltpu.sync_copy(dm_v, m_out.at[:, pl.ds(base, BPW)])


def _sc_gather(user, movie, actor_t, country_t, type_t,
               uflat, mflat, actor_table, country_table, type_table):
    emb = jax.ShapeDtypeStruct((B, D), jnp.float32)
    embT = jax.ShapeDtypeStruct((D, B), jnp.float32)
    run = pl.kernel(
        _sc_gather_body,
        out_type=(embT, embT, emb, emb, emb),
        mesh=plsc.VectorSubcoreMesh(core_axis_name="c", subcore_axis_name="s",
                                    num_cores=NC, num_subcores=NS),
        scratch_types=[
            pltpu.VMEM((BPW,), jnp.int32),
            pltpu.VMEM((BPW,), jnp.int32),
            pltpu.VMEM((10, BPW), jnp.int32),
            pltpu.VMEM((N_COUNTRY, BPW), jnp.int32),
            pltpu.VMEM((N_TYPE, BPW), jnp.int32),
            pltpu.VMEM((D, BPW), jnp.int32),
            pltpu.VMEM((D, BPW), jnp.int32),
            pltpu.VMEM((D, BPW), jnp.float32),
            pltpu.VMEM((D, BPW), jnp.float32),
            pltpu.VMEM((BPW, D), jnp.float32),
            pltpu.VMEM((BPW, D), jnp.float32),
            pltpu.VMEM((BPW, D), jnp.float32),
            pltpu.SemaphoreType.DMA,
            pltpu.SemaphoreType.DMA,
            pltpu.SemaphoreType.DMA,
            pltpu.SemaphoreType.DMA,
        ],
        compiler_params=pltpu.CompilerParams(use_tc_tiling_on_sc=False),
    )
    return run(user, movie, actor_t, country_t, type_t,
               uflat, mflat, actor_table, country_table, type_table)


def _detile_body(inp, out):
    out[...] = inp[...].reshape(8 * UCW)


def _detile(tT, kb):
    grid = (D // 8, kb)
    return pl.pallas_call(
        _detile_body,
        grid=grid,
        in_specs=[pl.BlockSpec((8, UCW), lambda d8, k: (d8, k))],
        out_specs=pl.BlockSpec((8 * UCW,), lambda d8, k: (d8 * kb + k,)),
        out_shape=jax.ShapeDtypeStruct((D * kb * UCW,), jnp.float32),
    )(tT)


def _mlp_body(uT, mT, a, c, t, w1, b1, w2, b2, w3, b3, out):
    f32 = jnp.float32
    dn0 = (((0,), (0,)), ((), ()))
    h = (lax.dot_general(uT[...], w1[0:D, :], dn0, preferred_element_type=f32)
         + lax.dot_general(mT[...], w1[D:2 * D, :], dn0, preferred_element_type=f32)
         + jnp.dot(a[...] * (1.0 / N_ACTOR), w1[2 * D:3 * D, :], preferred_element_type=f32)
         + jnp.dot(c[...] * (1.0 / N_COUNTRY), w1[3 * D:4 * D, :], preferred_element_type=f32)
         + jnp.dot(t[...] * (1.0 / N_TYPE), w1[4 * D:5 * D, :], preferred_element_type=f32)
         + b1[...])
    h = jnp.maximum(h, 0.0)
    h2 = jnp.maximum(jnp.dot(h, w2[...], preferred_element_type=f32) + b2[...], 0.0)
    out[...] = jnp.dot(h2, w3[...], preferred_element_type=f32) + b3[...]


def _mlp(uT, mT, a, c, t, W1, b1, W2, b2, W3, b3):
    BM = 2048
    grid = (B // BM,)
    emb_spec = pl.BlockSpec((BM, D), lambda i: (i, 0))
    embT_spec = pl.BlockSpec((D, BM), lambda i: (0, i))
    full = lambda s: pl.BlockSpec(s, lambda i: tuple(0 for _ in s))
    return pl.pallas_call(
        _mlp_body,
        grid=grid,
        in_specs=[embT_spec, embT_spec, emb_spec, emb_spec, emb_spec,
                  full((5 * D, 64)), full((64,)), full((64, 32)), full((32,)),
                  full((32, 1)), full((1,))],
        out_specs=pl.BlockSpec((BM, 1), lambda i: (i, 0)),
        out_shape=jax.ShapeDtypeStruct((B, 1), jnp.float32),
    )(uT, mT, a, c, t, W1, b1, W2, b2, W3, b3)


def kernel(user, movie, actor, country, movie_type,
           user_table, movie_table, actor_table, country_table, type_table,
           W1, b1, W2, b2, W3, b3):
    user = user.astype(jnp.int32)
    actor_t = actor.T
    country_t = country.T
    type_t = movie_type.T
    uflat = _detile(user_table.T, KB_U)
    mflat = _detile(movie_table.T, KB_M)
    u, m, a, c, t = _sc_gather(user, movie, actor_t, country_t, type_t,
                               uflat, mflat, actor_table,
                               country_table, type_table)
    y = _mlp(u, m, a, c, t, W1, b1, W2, b2, W3, b3)
    return jnp.squeeze(y, axis=-1)


# speedup vs baseline: 9.5466x; 1.0908x over previous
import functools

import jax
import jax.numpy as jnp
from jax import lax
from jax.experimental import pallas as pl
from jax.experimental.pallas import tpu as pltpu
from jax.experimental.pallas import tpu_sc as plsc

B = 16384
D = 32
NC, NS = 2, 16
NW = NC * NS
BPW = B // NW
N_ACTOR, N_COUNTRY, N_TYPE = 20, 4, 8
UCW = 1 << 16
KB_U = 16
KB_M = 2


def _scalar_offsets(idx, kb, dd):
    return ((((dd >> 3) * kb + (idx >> 16)) << 19)
            + ((dd & 7) << 16) + (idx & 0xFFFF))


def _sc_scalar_body(user_hbm, movie_hbm, uflat_hbm, mflat_hbm,
                    u_out, m_out,
                    uidx_v, midx_v, eu_v, em_v, du_v, dm_v, sem_s):
    wid = lax.axis_index("s") * NC + lax.axis_index("c")
    base = wid * BPW

    pltpu.sync_copy(user_hbm.at[pl.ds(base, BPW)], uidx_v)
    pltpu.sync_copy(movie_hbm.at[pl.ds(base, BPW)], midx_v)

    def gen_d(dd, carry):
        def gen_i(c, carry2):
            i16 = c * 16
            eu_v[dd, pl.ds(i16, 16)] = _scalar_offsets(
                uidx_v[pl.ds(i16, 16)], KB_U, dd)
            em_v[dd, pl.ds(i16, 16)] = _scalar_offsets(
                midx_v[pl.ds(i16, 16)], KB_M, dd)
            return carry2
        lax.fori_loop(0, BPW // 16, gen_i, 0)
        return carry
    lax.fori_loop(0, D, gen_d, 0)

    def fire(dd, carry):
        pltpu.async_copy(uflat_hbm.at[eu_v.at[dd]], du_v.at[dd], sem_s)
        pltpu.async_copy(mflat_hbm.at[em_v.at[dd]], dm_v.at[dd], sem_s)
        return carry
    lax.fori_loop(0, D, fire, 0)

    def drain(dd, carry):
        pltpu.make_async_copy(uflat_hbm.at[eu_v.at[0]], du_v.at[0], sem_s).wait()
        pltpu.make_async_copy(mflat_hbm.at[em_v.at[0]], dm_v.at[0], sem_s).wait()
        return carry
    lax.fori_loop(0, D, drain, 0)
    pltpu.sync_copy(du_v, u_out.at[:, pl.ds(base, BPW)])
    pltpu.sync_copy(dm_v, m_out.at[:, pl.ds(base, BPW)])


def _sc_scalar(user, movie, uflat, mflat):
    embT = jax.ShapeDtypeStruct((D, B), jnp.float32)
    run = pl.kernel(
        _sc_scalar_body,
        out_type=(embT, embT),
        mesh=plsc.VectorSubcoreMesh(core_axis_name="c", subcore_axis_name="s",
                                    num_cores=NC, num_subcores=NS),
        scratch_types=[
            pltpu.VMEM((BPW,), jnp.int32),
            pltpu.VMEM((BPW,), jnp.int32),
            pltpu.VMEM((D, BPW), jnp.int32),
            pltpu.VMEM((D, BPW), jnp.int32),
            pltpu.VMEM((D, BPW), jnp.float32),
            pltpu.VMEM((D, BPW), jnp.float32),
            pltpu.SemaphoreType.DMA,
        ],
        compiler_params=pltpu.CompilerParams(use_tc_tiling_on_sc=False),
    )
    return run(user, movie, uflat, mflat)


def _sc_pooled_body(actor_hbm, country_hbm, type_hbm,
                    at_hbm, ct_hbm, tt_hbm,
                    a_out, c_out, t_out,
                    aidx_v, cidx_v, tidx_v, acc_a, acc_c, acc_t,
                    sem_a, sem_c, sem_t):
    wid = lax.axis_index("s") * NC + lax.axis_index("c")
    base = wid * BPW

    pltpu.sync_copy(actor_hbm.at[pl.ds(0, 10), pl.ds(base, BPW)], aidx_v)
    pltpu.sync_copy(country_hbm.at[:, pl.ds(base, BPW)], cidx_v)
    pltpu.sync_copy(type_hbm.at[:, pl.ds(base, BPW)], tidx_v)

    def fire_p(table, idx_v, j, acc, sem, add):
        pltpu.async_copy(table.at[idx_v.at[j]], acc, sem, add=add)

    def wait_p(table, idx_v, acc, sem):
        pltpu.make_async_copy(table.at[idx_v.at[0]], acc, sem).wait()

    fire_p(at_hbm, aidx_v, 0, acc_a, sem_a, False)
    fire_p(ct_hbm, cidx_v, 0, acc_c, sem_c, False)
    fire_p(tt_hbm, tidx_v, 0, acc_t, sem_t, False)

    def step3(j, carry):
        wait_p(at_hbm, aidx_v, acc_a, sem_a)
        fire_p(at_hbm, aidx_v, j, acc_a, sem_a, True)
        wait_p(ct_hbm, cidx_v, acc_c, sem_c)
        fire_p(ct_hbm, cidx_v, j, acc_c, sem_c, True)
        wait_p(tt_hbm, tidx_v, acc_t, sem_t)
        fire_p(tt_hbm, tidx_v, j, acc_t, sem_t, True)
        return carry
    lax.fori_loop(1, N_COUNTRY, step3, 0)

    def step2(j, carry):
        wait_p(at_hbm, aidx_v, acc_a, sem_a)
        fire_p(at_hbm, aidx_v, j, acc_a, sem_a, True)
        wait_p(tt_hbm, tidx_v, acc_t, sem_t)
        fire_p(tt_hbm, tidx_v, j, acc_t, sem_t, True)
        return carry
    lax.fori_loop(N_COUNTRY, N_TYPE, step2, 0)

    def step1(j, carry):
        wait_p(at_hbm, aidx_v, acc_a, sem_a)
        fire_p(at_hbm, aidx_v, j, acc_a, sem_a, True)
        return carry
    lax.fori_loop(N_TYPE, 10, step1, 0)

    wait_p(at_hbm, aidx_v, acc_a, sem_a)
    pltpu.sync_copy(actor_hbm.at[pl.ds(10, 10), pl.ds(base, BPW)], aidx_v)
    fire_p(at_hbm, aidx_v, 0, acc_a, sem_a, True)

    def step1b(j, carry):
        wait_p(at_hbm, aidx_v, acc_a, sem_a)
        fire_p(at_hbm, aidx_v, j, acc_a, sem_a, True)
        return carry
    lax.fori_loop(1, 10, step1b, 0)

    wait_p(ct_hbm, cidx_v, acc_c, sem_c)
    pltpu.sync_copy(acc_c, c_out.at[pl.ds(base, BPW)])
    wait_p(tt_hbm, tidx_v, acc_t, sem_t)
    pltpu.sync_copy(acc_t, t_out.at[pl.ds(base, BPW)])
    wait_p(at_hbm, aidx_v, acc_a, sem_a)
    pltpu.sync_copy(acc_a, a_out.at[pl.ds(base, BPW)])


def _sc_pooled(actor_t, country_t, type_t,
               actor_table, country_table, type_table):
    emb = jax.ShapeDtypeStruct((B, D), jnp.float32)
    run = pl.kernel(
        _sc_pooled_body,
        out_type=(emb, emb, emb),
        mesh=plsc.VectorSubcoreMesh(core_axis_name="c", subcore_axis_name="s",
                                    num_cores=NC, num_subcores=NS),
        scratch_types=[
            pltpu.VMEM((10, BPW), jnp.int32),
            pltpu.VMEM((N_COUNTRY, BPW), jnp.int32),
            pltpu.VMEM((N_TYPE, BPW), jnp.int32),
            pltpu.VMEM((BPW, D), jnp.float32),
            pltpu.VMEM((BPW, D), jnp.float32),
            pltpu.VMEM((BPW, D), jnp.float32),
            pltpu.SemaphoreType.DMA,
            pltpu.SemaphoreType.DMA,
            pltpu.SemaphoreType.DMA,
        ],
        compiler_params=pltpu.CompilerParams(use_tc_tiling_on_sc=False),
    )
    return run(actor_t, country_t, type_t,
               actor_table, country_table, type_table)


def _detile_body(inp, out):
    out[...] = inp[...].reshape(8 * UCW)


def _detile(tT, kb):
    grid = (D // 8, kb)
    return pl.pallas_call(
        _detile_body,
        grid=grid,
        in_specs=[pl.BlockSpec((8, UCW), lambda d8, k: (d8, k))],
        out_specs=pl.BlockSpec((8 * UCW,), lambda d8, k: (d8 * kb + k,)),
        out_shape=jax.ShapeDtypeStruct((D * kb * UCW,), jnp.float32),
    )(tT)


def _mlp_body(uT, mT, a, c, t, w1, b1, w2, b2, w3, b3, out):
    f32 = jnp.float32
    dn0 = (((0,), (0,)), ((), ()))
    h = (lax.dot_general(uT[...], w1[0:D, :], dn0, preferred_element_type=f32)
         + lax.dot_general(mT[...], w1[D:2 * D, :], dn0, preferred_element_type=f32)
         + jnp.dot(a[...] * (1.0 / N_ACTOR), w1[2 * D:3 * D, :], preferred_element_type=f32)
         + jnp.dot(c[...] * (1.0 / N_COUNTRY), w1[3 * D:4 * D, :], preferred_element_type=f32)
         + jnp.dot(t[...] * (1.0 / N_TYPE), w1[4 * D:5 * D, :], preferred_element_type=f32)
         + b1[...])
    h = jnp.maximum(h, 0.0)
    h2 = jnp.maximum(jnp.dot(h, w2[...], preferred_element_type=f32) + b2[...], 0.0)
    out[...] = jnp.dot(h2, w3[...], preferred_element_type=f32) + b3[...]


def _mlp(uT, mT, a, c, t, W1, b1, W2, b2, W3, b3):
    BM = 2048
    grid = (B // BM,)
    emb_spec = pl.BlockSpec((BM, D), lambda i: (i, 0))
    embT_spec = pl.BlockSpec((D, BM), lambda i: (0, i))
    full = lambda s: pl.BlockSpec(s, lambda i: tuple(0 for _ in s))
    return pl.pallas_call(
        _mlp_body,
        grid=grid,
        in_specs=[embT_spec, embT_spec, emb_spec, emb_spec, emb_spec,
                  full((5 * D, 64)), full((64,)), full((64, 32)), full((32,)),
                  full((32, 1)), full((1,))],
        out_specs=pl.BlockSpec((BM, 1), lambda i: (i, 0)),
        out_shape=jax.ShapeDtypeStruct((B, 1), jnp.float32),
    )(uT, mT, a, c, t, W1, b1, W2, b2, W3, b3)


def kernel(user, movie, actor, country, movie_type,
           user_table, movie_table, actor_table, country_table, type_table,
           W1, b1, W2, b2, W3, b3):
    user = user.astype(jnp.int32)
    actor_t = actor.T
    country_t = country.T
    type_t = movie_type.T
    uflat = _detile(user_table.T, KB_U)
    mflat = _detile(movie_table.T, KB_M)
    a, c, t = _sc_pooled(actor_t, country_t, type_t,
                         actor_table, country_table, type_table)
    u, m = _sc_scalar(user, movie, uflat, mflat)
    y = _mlp(u, m, a, c, t, W1, b1, W2, b2, W3, b3)
    return jnp.squeeze(y, axis=-1)


# speedup vs baseline: 9.6626x; 1.0122x over previous
import functools

import jax
import jax.numpy as jnp
from jax import lax
from jax.experimental import pallas as pl
from jax.experimental.pallas import tpu as pltpu
from jax.experimental.pallas import tpu_sc as plsc

B = 16384
D = 32
NC, NS = 2, 16
NW = NC * NS
BPW = B // NW
N_ACTOR, N_COUNTRY, N_TYPE = 20, 4, 8
UCW = 1 << 16
KB_U = 16
KB_M = 2


def _scalar_offsets(idx, kb, dd):
    return ((((dd >> 3) * kb + (idx >> 16)) << 19)
            + ((dd & 7) << 16) + (idx & 0xFFFF))


def _sc_scalar_body(user_hbm, movie_hbm, uflat_hbm, mflat_hbm,
                    u_out, m_out,
                    uidx_v, midx_v, eu_v, em_v, du_v, dm_v, sem_s):
    wid = lax.axis_index("s") * NC + lax.axis_index("c")
    base = wid * BPW

    pltpu.sync_copy(user_hbm.at[pl.ds(base, BPW)], uidx_v)
    pltpu.sync_copy(movie_hbm.at[pl.ds(base, BPW)], midx_v)

    def gen_d(dd, carry):
        def gen_i(c, carry2):
            i16 = c * 16
            eu_v[dd, pl.ds(i16, 16)] = _scalar_offsets(
                uidx_v[pl.ds(i16, 16)], KB_U, dd)
            em_v[dd, pl.ds(i16, 16)] = _scalar_offsets(
                midx_v[pl.ds(i16, 16)], KB_M, dd)
            return carry2
        lax.fori_loop(0, BPW // 16, gen_i, 0)
        return carry
    lax.fori_loop(0, D, gen_d, 0)

    def fire(dd, carry):
        pltpu.async_copy(uflat_hbm.at[eu_v.at[dd]], du_v.at[dd], sem_s)
        pltpu.async_copy(mflat_hbm.at[em_v.at[dd]], dm_v.at[dd], sem_s)
        return carry
    lax.fori_loop(0, D, fire, 0)

    def drain(dd, carry):
        pltpu.make_async_copy(uflat_hbm.at[eu_v.at[0]], du_v.at[0], sem_s).wait()
        pltpu.make_async_copy(mflat_hbm.at[em_v.at[0]], dm_v.at[0], sem_s).wait()
        return carry
    lax.fori_loop(0, D, drain, 0)
    pltpu.sync_copy(du_v, u_out.at[:, pl.ds(base, BPW)])
    pltpu.sync_copy(dm_v, m_out.at[:, pl.ds(base, BPW)])


def _sc_scalar(user, movie, uflat, mflat):
    embT = jax.ShapeDtypeStruct((D, B), jnp.float32)
    run = pl.kernel(
        _sc_scalar_body,
        out_type=(embT, embT),
        mesh=plsc.VectorSubcoreMesh(core_axis_name="c", subcore_axis_name="s",
                                    num_cores=NC, num_subcores=NS),
        scratch_types=[
            pltpu.VMEM((BPW,), jnp.int32),
            pltpu.VMEM((BPW,), jnp.int32),
            pltpu.VMEM((D, BPW), jnp.int32),
            pltpu.VMEM((D, BPW), jnp.int32),
            pltpu.VMEM((D, BPW), jnp.float32),
            pltpu.VMEM((D, BPW), jnp.float32),
            pltpu.SemaphoreType.DMA,
        ],
        compiler_params=pltpu.CompilerParams(use_tc_tiling_on_sc=False),
    )
    return run(user, movie, uflat, mflat)


def _sc_pooled_body(actor_hbm, country_hbm, type_hbm,
                    at_hbm, ct_hbm, tt_hbm,
                    a_out, c_out, t_out,
                    aidx_v, cidx_v, tidx_v, acc_a, acc_a2, acc_c, acc_t,
                    sem_a, sem_a2, sem_c, sem_t):
    wid = lax.axis_index("s") * NC + lax.axis_index("c")
    base = wid * BPW

    pltpu.sync_copy(actor_hbm.at[:, pl.ds(base, BPW)], aidx_v)
    pltpu.sync_copy(country_hbm.at[:, pl.ds(base, BPW)], cidx_v)
    pltpu.sync_copy(type_hbm.at[:, pl.ds(base, BPW)], tidx_v)

    def fire_p(table, idx_v, j, acc, sem, add):
        pltpu.async_copy(table.at[idx_v.at[j]], acc, sem, add=add)

    def wait_p(table, idx_v, acc, sem):
        pltpu.make_async_copy(table.at[idx_v.at[0]], acc, sem).wait()

    fire_p(at_hbm, aidx_v, 0, acc_a, sem_a, False)
    fire_p(at_hbm, aidx_v, 1, acc_a2, sem_a2, False)
    fire_p(ct_hbm, cidx_v, 0, acc_c, sem_c, False)
    fire_p(tt_hbm, tidx_v, 0, acc_t, sem_t, False)

    def step4(j, carry):
        wait_p(at_hbm, aidx_v, acc_a, sem_a)
        fire_p(at_hbm, aidx_v, 2 * j, acc_a, sem_a, True)
        wait_p(at_hbm, aidx_v, acc_a2, sem_a2)
        fire_p(at_hbm, aidx_v, 2 * j + 1, acc_a2, sem_a2, True)
        wait_p(ct_hbm, cidx_v, acc_c, sem_c)
        fire_p(ct_hbm, cidx_v, j, acc_c, sem_c, True)
        wait_p(tt_hbm, tidx_v, acc_t, sem_t)
        fire_p(tt_hbm, tidx_v, j, acc_t, sem_t, True)
        return carry
    lax.fori_loop(1, N_COUNTRY, step4, 0)

    def step3(j, carry):
        wait_p(at_hbm, aidx_v, acc_a, sem_a)
        fire_p(at_hbm, aidx_v, 2 * j, acc_a, sem_a, True)
        wait_p(at_hbm, aidx_v, acc_a2, sem_a2)
        fire_p(at_hbm, aidx_v, 2 * j + 1, acc_a2, sem_a2, True)
        wait_p(tt_hbm, tidx_v, acc_t, sem_t)
        fire_p(tt_hbm, tidx_v, j, acc_t, sem_t, True)
        return carry
    lax.fori_loop(N_COUNTRY, N_TYPE, step3, 0)

    def step2(j, carry):
        wait_p(at_hbm, aidx_v, acc_a, sem_a)
        fire_p(at_hbm, aidx_v, 2 * j, acc_a, sem_a, True)
        wait_p(at_hbm, aidx_v, acc_a2, sem_a2)
        fire_p(at_hbm, aidx_v, 2 * j + 1, acc_a2, sem_a2, True)
        return carry
    lax.fori_loop(N_TYPE, N_ACTOR // 2, step2, 0)

    wait_p(ct_hbm, cidx_v, acc_c, sem_c)
    pltpu.sync_copy(acc_c, c_out.at[pl.ds(base, BPW)])
    wait_p(tt_hbm, tidx_v, acc_t, sem_t)
    pltpu.sync_copy(acc_t, t_out.at[pl.ds(base, BPW)])
    wait_p(at_hbm, aidx_v, acc_a, sem_a)
    wait_p(at_hbm, aidx_v, acc_a2, sem_a2)

    def merge(i, carry):
        r = i >> 1
        c16 = (i & 1) * 16
        acc_a[r, pl.ds(c16, 16)] = (acc_a[r, pl.ds(c16, 16)]
                                    + acc_a2[r, pl.ds(c16, 16)])
        return carry
    lax.fori_loop(0, BPW * 2, merge, 0)
    pltpu.sync_copy(acc_a, a_out.at[pl.ds(base, BPW)])


def _sc_pooled(actor_t, country_t, type_t,
               actor_table, country_table, type_table):
    emb = jax.ShapeDtypeStruct((B, D), jnp.float32)
    run = pl.kernel(
        _sc_pooled_body,
        out_type=(emb, emb, emb),
        mesh=plsc.VectorSubcoreMesh(core_axis_name="c", subcore_axis_name="s",
                                    num_cores=NC, num_subcores=NS),
        scratch_types=[
            pltpu.VMEM((N_ACTOR, BPW), jnp.int32),
            pltpu.VMEM((N_COUNTRY, BPW), jnp.int32),
            pltpu.VMEM((N_TYPE, BPW), jnp.int32),
            pltpu.VMEM((BPW, D), jnp.float32),
            pltpu.VMEM((BPW, D), jnp.float32),
            pltpu.VMEM((BPW, D), jnp.float32),
            pltpu.VMEM((BPW, D), jnp.float32),
            pltpu.SemaphoreType.DMA,
            pltpu.SemaphoreType.DMA,
            pltpu.SemaphoreType.DMA,
            pltpu.SemaphoreType.DMA,
        ],
        compiler_params=pltpu.CompilerParams(use_tc_tiling_on_sc=False),
    )
    return run(actor_t, country_t, type_t,
               actor_table, country_table, type_table)


def _detile_body(inp, out):
    out[...] = inp[...].reshape(8 * UCW)


def _detile(tT, kb):
    grid = (D // 8, kb)
    return pl.pallas_call(
        _detile_body,
        grid=grid,
        in_specs=[pl.BlockSpec((8, UCW), lambda d8, k: (d8, k))],
        out_specs=pl.BlockSpec((8 * UCW,), lambda d8, k: (d8 * kb + k,)),
        out_shape=jax.ShapeDtypeStruct((D * kb * UCW,), jnp.float32),
    )(tT)


def _mlp_body(uT, mT, a, c, t, w1, b1, w2, b2, w3, b3, out):
    f32 = jnp.float32
    dn0 = (((0,), (0,)), ((), ()))
    h = (lax.dot_general(uT[...], w1[0:D, :], dn0, preferred_element_type=f32)
         + lax.dot_general(mT[...], w1[D:2 * D, :], dn0, preferred_element_type=f32)
         + jnp.dot(a[...] * (1.0 / N_ACTOR), w1[2 * D:3 * D, :], preferred_element_type=f32)
         + jnp.dot(c[...] * (1.0 / N_COUNTRY), w1[3 * D:4 * D, :], preferred_element_type=f32)
         + jnp.dot(t[...] * (1.0 / N_TYPE), w1[4 * D:5 * D, :], preferred_element_type=f32)
         + b1[...])
    h = jnp.maximum(h, 0.0)
    h2 = jnp.maximum(jnp.dot(h, w2[...], preferred_element_type=f32) + b2[...], 0.0)
    out[...] = jnp.dot(h2, w3[...], preferred_element_type=f32) + b3[...]


def _mlp(uT, mT, a, c, t, W1, b1, W2, b2, W3, b3):
    BM = 2048
    grid = (B // BM,)
    emb_spec = pl.BlockSpec((BM, D), lambda i: (i, 0))
    embT_spec = pl.BlockSpec((D, BM), lambda i: (0, i))
    full = lambda s: pl.BlockSpec(s, lambda i: tuple(0 for _ in s))
    return pl.pallas_call(
        _mlp_body,
        grid=grid,
        in_specs=[embT_spec, embT_spec, emb_spec, emb_spec, emb_spec,
                  full((5 * D, 64)), full((64,)), full((64, 32)), full((32,)),
                  full((32, 1)), full((1,))],
        out_specs=pl.BlockSpec((BM, 1), lambda i: (i, 0)),
        out_shape=jax.ShapeDtypeStruct((B, 1), jnp.float32),
    )(uT, mT, a, c, t, W1, b1, W2, b2, W3, b3)


def kernel(user, movie, actor, country, movie_type,
           user_table, movie_table, actor_table, country_table, type_table,
           W1, b1, W2, b2, W3, b3):
    user = user.astype(jnp.int32)
    actor_t = actor.T
    country_t = country.T
    type_t = movie_type.T
    uflat = _detile(user_table.T, KB_U)
    mflat = _detile(movie_table.T, KB_M)
    a, c, t = _sc_pooled(actor_t, country_t, type_t,
                         actor_table, country_table, type_table)
    u, m = _sc_scalar(user, movie, uflat, mflat)
    y = _mlp(u, m, a, c, t, W1, b1, W2, b2, W3, b3)
    return jnp.squeeze(y, axis=-1)


# speedup vs baseline: 10.1101x; 1.0463x over previous
import functools

import jax
import jax.numpy as jnp
from jax import lax
from jax.experimental import pallas as pl
from jax.experimental.pallas import tpu as pltpu
from jax.experimental.pallas import tpu_sc as plsc

B = 16384
D = 32
NC, NS = 2, 16
NW = NC * NS
BPW = B // NW
N_ACTOR, N_COUNTRY, N_TYPE = 20, 4, 8
UCB = 17
UCW = 1 << UCB
KB_U = 8
KB_M = 1


def _scalar_offsets(idx, kb, dd):
    return ((((dd >> 3) * kb + (idx >> UCB)) << (UCB + 3))
            + ((dd & 7) << UCB) + (idx & (UCW - 1)))


def _sc_scalar_body(user_hbm, movie_hbm, uflat_hbm, mflat_hbm,
                    u_out, m_out,
                    uidx_v, midx_v, eu_v, em_v, du_v, dm_v, sem_s):
    wid = lax.axis_index("s") * NC + lax.axis_index("c")
    base = wid * BPW

    pltpu.sync_copy(user_hbm.at[pl.ds(base, BPW)], uidx_v)
    pltpu.sync_copy(movie_hbm.at[pl.ds(base, BPW)], midx_v)

    def gen_d(dd, carry):
        def gen_i(c, carry2):
            i16 = c * 16
            eu_v[dd, pl.ds(i16, 16)] = _scalar_offsets(
                uidx_v[pl.ds(i16, 16)], KB_U, dd)
            em_v[dd, pl.ds(i16, 16)] = _scalar_offsets(
                midx_v[pl.ds(i16, 16)], KB_M, dd)
            return carry2
        lax.fori_loop(0, BPW // 16, gen_i, 0)
        return carry
    lax.fori_loop(0, D, gen_d, 0)

    def fire(dd, carry):
        pltpu.async_copy(uflat_hbm.at[eu_v.at[dd]], du_v.at[dd], sem_s)
        pltpu.async_copy(mflat_hbm.at[em_v.at[dd]], dm_v.at[dd], sem_s)
        return carry
    lax.fori_loop(0, D, fire, 0)

    def drain(dd, carry):
        pltpu.make_async_copy(uflat_hbm.at[eu_v.at[0]], du_v.at[0], sem_s).wait()
        pltpu.make_async_copy(mflat_hbm.at[em_v.at[0]], dm_v.at[0], sem_s).wait()
        return carry
    lax.fori_loop(0, D, drain, 0)
    pltpu.sync_copy(du_v, u_out.at[:, pl.ds(base, BPW)])
    pltpu.sync_copy(dm_v, m_out.at[:, pl.ds(base, BPW)])


def _sc_scalar(user, movie, uflat, mflat):
    embT = jax.ShapeDtypeStruct((D, B), jnp.float32)
    run = pl.kernel(
        _sc_scalar_body,
        out_type=(embT, embT),
        mesh=plsc.VectorSubcoreMesh(core_axis_name="c", subcore_axis_name="s",
                                    num_cores=NC, num_subcores=NS),
        scratch_types=[
            pltpu.VMEM((BPW,), jnp.int32),
            pltpu.VMEM((BPW,), jnp.int32),
            pltpu.VMEM((D, BPW), jnp.int32),
            pltpu.VMEM((D, BPW), jnp.int32),
            pltpu.VMEM((D, BPW), jnp.float32),
            pltpu.VMEM((D, BPW), jnp.float32),
            pltpu.SemaphoreType.DMA,
        ],
        compiler_params=pltpu.CompilerParams(use_tc_tiling_on_sc=False),
    )
    return run(user, movie, uflat, mflat)


def _sc_pooled_body(actor_hbm, country_hbm, type_hbm,
                    at_hbm, ct_hbm, tt_hbm,
                    a_out, c_out, t_out,
                    aidx_v, cidx_v, tidx_v, acc_a, acc_a2, acc_c, acc_t,
                    sem_a, sem_a2, sem_c, sem_t):
    wid = lax.axis_index("s") * NC + lax.axis_index("c")
    base = wid * BPW

    pltpu.sync_copy(actor_hbm.at[:, pl.ds(base, BPW)], aidx_v)
    pltpu.sync_copy(country_hbm.at[:, pl.ds(base, BPW)], cidx_v)
    pltpu.sync_copy(type_hbm.at[:, pl.ds(base, BPW)], tidx_v)

    def fire_p(table, idx_v, j, acc, sem, add):
        pltpu.async_copy(table.at[idx_v.at[j]], acc, sem, add=add)

    def wait_p(table, idx_v, acc, sem):
        pltpu.make_async_copy(table.at[idx_v.at[0]], acc, sem).wait()

    fire_p(at_hbm, aidx_v, 0, acc_a, sem_a, False)
    fire_p(at_hbm, aidx_v, 1, acc_a2, sem_a2, False)
    fire_p(ct_hbm, cidx_v, 0, acc_c, sem_c, False)
    fire_p(tt_hbm, tidx_v, 0, acc_t, sem_t, False)

    def step4(j, carry):
        wait_p(at_hbm, aidx_v, acc_a, sem_a)
        fire_p(at_hbm, aidx_v, 2 * j, acc_a, sem_a, True)
        wait_p(at_hbm, aidx_v, acc_a2, sem_a2)
        fire_p(at_hbm, aidx_v, 2 * j + 1, acc_a2, sem_a2, True)
        wait_p(ct_hbm, cidx_v, acc_c, sem_c)
        fire_p(ct_hbm, cidx_v, j, acc_c, sem_c, True)
        wait_p(tt_hbm, tidx_v, acc_t, sem_t)
        fire_p(tt_hbm, tidx_v, j, acc_t, sem_t, True)
        return carry
    lax.fori_loop(1, N_COUNTRY, step4, 0)

    def step3(j, carry):
        wait_p(at_hbm, aidx_v, acc_a, sem_a)
        fire_p(at_hbm, aidx_v, 2 * j, acc_a, sem_a, True)
        wait_p(at_hbm, aidx_v, acc_a2, sem_a2)
        fire_p(at_hbm, aidx_v, 2 * j + 1, acc_a2, sem_a2, True)
        wait_p(tt_hbm, tidx_v, acc_t, sem_t)
        fire_p(tt_hbm, tidx_v, j, acc_t, sem_t, True)
        return carry
    lax.fori_loop(N_COUNTRY, N_TYPE, step3, 0)

    def step2(j, carry):
        wait_p(at_hbm, aidx_v, acc_a, sem_a)
        fire_p(at_hbm, aidx_v, 2 * j, acc_a, sem_a, True)
        wait_p(at_hbm, aidx_v, acc_a2, sem_a2)
        fire_p(at_hbm, aidx_v, 2 * j + 1, acc_a2, sem_a2, True)
        return carry
    lax.fori_loop(N_TYPE, N_ACTOR // 2, step2, 0)

    wait_p(ct_hbm, cidx_v, acc_c, sem_c)
    pltpu.sync_copy(acc_c, c_out.at[pl.ds(base, BPW)])
    wait_p(tt_hbm, tidx_v, acc_t, sem_t)
    pltpu.sync_copy(acc_t, t_out.at[pl.ds(base, BPW)])
    wait_p(at_hbm, aidx_v, acc_a, sem_a)
    wait_p(at_hbm, aidx_v, acc_a2, sem_a2)

    def merge(i, carry):
        r = i >> 1
        c16 = (i & 1) * 16
        acc_a[r, pl.ds(c16, 16)] = (acc_a[r, pl.ds(c16, 16)]
                                    + acc_a2[r, pl.ds(c16, 16)])
        return carry
    lax.fori_loop(0, BPW * 2, merge, 0)
    pltpu.sync_copy(acc_a, a_out.at[pl.ds(base, BPW)])


def _sc_pooled(actor_t, country_t, type_t,
               actor_table, country_table, type_table):
    emb = jax.ShapeDtypeStruct((B, D), jnp.float32)
    run = pl.kernel(
        _sc_pooled_body,
        out_type=(emb, emb, emb),
        mesh=plsc.VectorSubcoreMesh(core_axis_name="c", subcore_axis_name="s",
                                    num_cores=NC, num_subcores=NS),
        scratch_types=[
            pltpu.VMEM((N_ACTOR, BPW), jnp.int32),
            pltpu.VMEM((N_COUNTRY, BPW), jnp.int32),
            pltpu.VMEM((N_TYPE, BPW), jnp.int32),
            pltpu.VMEM((BPW, D), jnp.float32),
            pltpu.VMEM((BPW, D), jnp.float32),
            pltpu.VMEM((BPW, D), jnp.float32),
            pltpu.VMEM((BPW, D), jnp.float32),
            pltpu.SemaphoreType.DMA,
            pltpu.SemaphoreType.DMA,
            pltpu.SemaphoreType.DMA,
            pltpu.SemaphoreType.DMA,
        ],
        compiler_params=pltpu.CompilerParams(use_tc_tiling_on_sc=False),
    )
    return run(actor_t, country_t, type_t,
               actor_table, country_table, type_table)


def _detile_body(inp, out):
    out[...] = inp[...].reshape(8 * UCW)


def _detile(tT, kb):
    grid = (D // 8, kb)
    return pl.pallas_call(
        _detile_body,
        grid=grid,
        in_specs=[pl.BlockSpec((8, UCW), lambda d8, k: (d8, k))],
        out_specs=pl.BlockSpec((8 * UCW,), lambda d8, k: (d8 * kb + k,)),
        out_shape=jax.ShapeDtypeStruct((D * kb * UCW,), jnp.float32),
    )(tT)


def _mlp_body(uT, mT, a, c, t, w1, b1, w2, b2, w3, b3, out):
    f32 = jnp.float32
    dn0 = (((0,), (0,)), ((), ()))
    h = (lax.dot_general(uT[...], w1[0:D, :], dn0, preferred_element_type=f32)
         + lax.dot_general(mT[...], w1[D:2 * D, :], dn0, preferred_element_type=f32)
         + jnp.dot(a[...] * (1.0 / N_ACTOR), w1[2 * D:3 * D, :], preferred_element_type=f32)
         + jnp.dot(c[...] * (1.0 / N_COUNTRY), w1[3 * D:4 * D, :], preferred_element_type=f32)
         + jnp.dot(t[...] * (1.0 / N_TYPE), w1[4 * D:5 * D, :], preferred_element_type=f32)
         + b1[...])
    h = jnp.maximum(h, 0.0)
    h2 = jnp.maximum(jnp.dot(h, w2[...], preferred_element_type=f32) + b2[...], 0.0)
    out[...] = jnp.dot(h2, w3[...], preferred_element_type=f32) + b3[...]


def _mlp(uT, mT, a, c, t, W1, b1, W2, b2, W3, b3):
    BM = 2048
    grid = (B // BM,)
    emb_spec = pl.BlockSpec((BM, D), lambda i: (i, 0))
    embT_spec = pl.BlockSpec((D, BM), lambda i: (0, i))
    full = lambda s: pl.BlockSpec(s, lambda i: tuple(0 for _ in s))
    return pl.pallas_call(
        _mlp_body,
        grid=grid,
        in_specs=[embT_spec, embT_spec, emb_spec, emb_spec, emb_spec,
                  full((5 * D, 64)), full((64,)), full((64, 32)), full((32,)),
                  full((32, 1)), full((1,))],
        out_specs=pl.BlockSpec((BM, 1), lambda i: (i, 0)),
        out_shape=jax.ShapeDtypeStruct((B, 1), jnp.float32),
    )(uT, mT, a, c, t, W1, b1, W2, b2, W3, b3)


def kernel(user, movie, actor, country, movie_type,
           user_table, movie_table, actor_table, country_table, type_table,
           W1, b1, W2, b2, W3, b3):
    user = user.astype(jnp.int32)
    actor_t = actor.T
    country_t = country.T
    type_t = movie_type.T
    uflat = _detile(user_table.T, KB_U)
    mflat = _detile(movie_table.T, KB_M)
    a, c, t = _sc_pooled(actor_t, country_t, type_t,
                         actor_table, country_table, type_table)
    u, m = _sc_scalar(user, movie, uflat, mflat)
    y = _mlp(u, m, a, c, t, W1, b1, W2, b2, W3, b3)
    return jnp.squeeze(y, axis=-1)


# speedup vs baseline: 10.1355x; 1.0025x over previous
import functools

import jax
import jax.numpy as jnp
from jax import lax
from jax.experimental import pallas as pl
from jax.experimental.pallas import tpu as pltpu
from jax.experimental.pallas import tpu_sc as plsc

B = 16384
D = 32
NC, NS = 2, 16
NW = NC * NS
BPW = B // NW
N_ACTOR, N_COUNTRY, N_TYPE = 20, 4, 8
UCB = 17
UCW = 1 << UCB
KB_U = 8
KB_M = 1


def _scalar_offsets(idx, kb, dd):
    return ((((dd >> 3) * kb + (idx >> UCB)) << (UCB + 3))
            + ((dd & 7) << UCB) + (idx & (UCW - 1)))


def _sc_scalar_body(user_hbm, movie_hbm, uflat_hbm, mflat_hbm,
                    u_out, m_out,
                    uidx_v, midx_v, eu_v, em_v, du_v, dm_v, sem_s):
    wid = lax.axis_index("s") * NC + lax.axis_index("c")
    base = wid * BPW

    pltpu.sync_copy(user_hbm.at[pl.ds(base, BPW)], uidx_v)
    pltpu.sync_copy(movie_hbm.at[pl.ds(base, BPW)], midx_v)

    def gen_d(dd, carry):
        def gen_i(c, carry2):
            i16 = c * 16
            eu_v[dd, pl.ds(i16, 16)] = _scalar_offsets(
                uidx_v[pl.ds(i16, 16)], KB_U, dd)
            em_v[dd, pl.ds(i16, 16)] = _scalar_offsets(
                midx_v[pl.ds(i16, 16)], KB_M, dd)
            return carry2
        lax.fori_loop(0, BPW // 16, gen_i, 0)
        return carry
    lax.fori_loop(0, D, gen_d, 0)

    def fire(dd, carry):
        pltpu.async_copy(uflat_hbm.at[eu_v.at[dd]], du_v.at[dd], sem_s)
        pltpu.async_copy(mflat_hbm.at[em_v.at[dd]], dm_v.at[dd], sem_s)
        return carry
    lax.fori_loop(0, D, fire, 0)

    def drain(dd, carry):
        pltpu.make_async_copy(uflat_hbm.at[eu_v.at[0]], du_v.at[0], sem_s).wait()
        pltpu.make_async_copy(mflat_hbm.at[em_v.at[0]], dm_v.at[0], sem_s).wait()
        return carry
    lax.fori_loop(0, D, drain, 0)
    pltpu.sync_copy(du_v, u_out.at[:, pl.ds(base, BPW)])
    pltpu.sync_copy(dm_v, m_out.at[:, pl.ds(base, BPW)])


def _sc_scalar(user, movie, uflat, mflat):
    embT = jax.ShapeDtypeStruct((D, B), jnp.float32)
    run = pl.kernel(
        _sc_scalar_body,
        out_type=(embT, embT),
        mesh=plsc.VectorSubcoreMesh(core_axis_name="c", subcore_axis_name="s",
                                    num_cores=NC, num_subcores=NS),
        scratch_types=[
            pltpu.VMEM((BPW,), jnp.int32),
            pltpu.VMEM((BPW,), jnp.int32),
            pltpu.VMEM((D, BPW), jnp.int32),
            pltpu.VMEM((D, BPW), jnp.int32),
            pltpu.VMEM((D, BPW), jnp.float32),
            pltpu.VMEM((D, BPW), jnp.float32),
            pltpu.SemaphoreType.DMA,
        ],
        compiler_params=pltpu.CompilerParams(use_tc_tiling_on_sc=False),
    )
    return run(user, movie, uflat, mflat)


def _sc_pooled_body(actor_hbm, country_hbm, type_hbm,
                    at_hbm, ct_hbm, tt_hbm,
                    a_out, c_out, t_out,
                    aidx_v, cidx_v, tidx_v, acc_a, acc_a2, acc_c, acc_t,
                    sem_a, sem_a2, sem_c, sem_t):
    wid = lax.axis_index("s") * NC + lax.axis_index("c")
    base = wid * BPW

    pltpu.sync_copy(actor_hbm.at[:, pl.ds(base, BPW)], aidx_v)
    pltpu.sync_copy(country_hbm.at[:, pl.ds(base, BPW)], cidx_v)
    pltpu.sync_copy(type_hbm.at[:, pl.ds(base, BPW)], tidx_v)

    def fire_p(table, idx_v, j, acc, sem, add):
        pltpu.async_copy(table.at[idx_v.at[j]], acc, sem, add=add)

    def wait_p(table, idx_v, acc, sem):
        pltpu.make_async_copy(table.at[idx_v.at[0]], acc, sem).wait()

    fire_p(at_hbm, aidx_v, 0, acc_a, sem_a, False)
    fire_p(at_hbm, aidx_v, 1, acc_a2, sem_a2, False)
    fire_p(ct_hbm, cidx_v, 0, acc_c, sem_c, False)
    fire_p(tt_hbm, tidx_v, 0, acc_t, sem_t, False)

    def step4(j, carry):
        wait_p(at_hbm, aidx_v, acc_a, sem_a)
        fire_p(at_hbm, aidx_v, 2 * j, acc_a, sem_a, True)
        wait_p(at_hbm, aidx_v, acc_a2, sem_a2)
        fire_p(at_hbm, aidx_v, 2 * j + 1, acc_a2, sem_a2, True)
        wait_p(ct_hbm, cidx_v, acc_c, sem_c)
        fire_p(ct_hbm, cidx_v, j, acc_c, sem_c, True)
        wait_p(tt_hbm, tidx_v, acc_t, sem_t)
        fire_p(tt_hbm, tidx_v, j, acc_t, sem_t, True)
        return carry
    lax.fori_loop(1, N_COUNTRY, step4, 0)

    def step3(j, carry):
        wait_p(at_hbm, aidx_v, acc_a, sem_a)
        fire_p(at_hbm, aidx_v, 2 * j, acc_a, sem_a, True)
        wait_p(at_hbm, aidx_v, acc_a2, sem_a2)
        fire_p(at_hbm, aidx_v, 2 * j + 1, acc_a2, sem_a2, True)
        wait_p(tt_hbm, tidx_v, acc_t, sem_t)
        fire_p(tt_hbm, tidx_v, j, acc_t, sem_t, True)
        return carry
    lax.fori_loop(N_COUNTRY, N_TYPE, step3, 0)

    def step2(j, carry):
        wait_p(at_hbm, aidx_v, acc_a, sem_a)
        fire_p(at_hbm, aidx_v, 2 * j, acc_a, sem_a, True)
        wait_p(at_hbm, aidx_v, acc_a2, sem_a2)
        fire_p(at_hbm, aidx_v, 2 * j + 1, acc_a2, sem_a2, True)
        return carry
    lax.fori_loop(N_TYPE, N_ACTOR // 2, step2, 0)

    wait_p(ct_hbm, cidx_v, acc_c, sem_c)
    pltpu.sync_copy(acc_c, c_out.at[pl.ds(base, BPW)])
    wait_p(tt_hbm, tidx_v, acc_t, sem_t)
    pltpu.sync_copy(acc_t, t_out.at[pl.ds(base, BPW)])
    wait_p(at_hbm, aidx_v, acc_a, sem_a)
    wait_p(at_hbm, aidx_v, acc_a2, sem_a2)

    def merge(i, carry):
        r = i >> 1
        c16 = (i & 1) * 16
        acc_a[r, pl.ds(c16, 16)] = (acc_a[r, pl.ds(c16, 16)]
                                    + acc_a2[r, pl.ds(c16, 16)])
        return carry
    lax.fori_loop(0, BPW * 2, merge, 0)
    pltpu.sync_copy(acc_a, a_out.at[pl.ds(base, BPW)])


def _sc_pooled(actor_t, country_t, type_t,
               actor_table, country_table, type_table):
    emb = jax.ShapeDtypeStruct((B, D), jnp.float32)
    run = pl.kernel(
        _sc_pooled_body,
        out_type=(emb, emb, emb),
        mesh=plsc.VectorSubcoreMesh(core_axis_name="c", subcore_axis_name="s",
                                    num_cores=NC, num_subcores=NS),
        scratch_types=[
            pltpu.VMEM((N_ACTOR, BPW), jnp.int32),
            pltpu.VMEM((N_COUNTRY, BPW), jnp.int32),
            pltpu.VMEM((N_TYPE, BPW), jnp.int32),
            pltpu.VMEM((BPW, D), jnp.float32),
            pltpu.VMEM((BPW, D), jnp.float32),
            pltpu.VMEM((BPW, D), jnp.float32),
            pltpu.VMEM((BPW, D), jnp.float32),
            pltpu.SemaphoreType.DMA,
            pltpu.SemaphoreType.DMA,
            pltpu.SemaphoreType.DMA,
            pltpu.SemaphoreType.DMA,
        ],
        compiler_params=pltpu.CompilerParams(use_tc_tiling_on_sc=False),
    )
    return run(actor_t, country_t, type_t,
               actor_table, country_table, type_table)


def _detile_body(inp, out):
    out[...] = inp[...].reshape(8 * UCW)


def _detile(tT, kb):
    grid = (D // 8, kb)
    return pl.pallas_call(
        _detile_body,
        grid=grid,
        in_specs=[pl.BlockSpec((8, UCW), lambda d8, k: (d8, k))],
        out_specs=pl.BlockSpec((8 * UCW,), lambda d8, k: (d8 * kb + k,)),
        out_shape=jax.ShapeDtypeStruct((D * kb * UCW,), jnp.float32),
    )(tT)


def _mlp_body(uT, mT, a, c, t, w1, b1, w2, b2, w3, b3, out):
    f32 = jnp.float32
    dn0 = (((0,), (0,)), ((), ()))
    h = (lax.dot_general(uT[...], w1[0:D, :], dn0, preferred_element_type=f32)
         + lax.dot_general(mT[...], w1[D:2 * D, :], dn0, preferred_element_type=f32)
         + jnp.dot(a[...] * (1.0 / N_ACTOR), w1[2 * D:3 * D, :], preferred_element_type=f32)
         + jnp.dot(c[...] * (1.0 / N_COUNTRY), w1[3 * D:4 * D, :], preferred_element_type=f32)
         + jnp.dot(t[...] * (1.0 / N_TYPE), w1[4 * D:5 * D, :], preferred_element_type=f32)
         + b1[...])
    h = jnp.maximum(h, 0.0)
    h2 = jnp.maximum(jnp.dot(h, w2[...], preferred_element_type=f32) + b2[...], 0.0)
    out[...] = jnp.dot(h2, w3[...], preferred_element_type=f32) + b3[...]


def _mlp(uT, mT, a, c, t, W1, b1, W2, b2, W3, b3):
    BM = 2048
    grid = (B // BM,)
    emb_spec = pl.BlockSpec((BM, D), lambda i: (i, 0))
    embT_spec = pl.BlockSpec((D, BM), lambda i: (0, i))
    full = lambda s: pl.BlockSpec(s, lambda i: tuple(0 for _ in s))
    return pl.pallas_call(
        _mlp_body,
        grid=grid,
        in_specs=[embT_spec, embT_spec, emb_spec, emb_spec, emb_spec,
                  full((5 * D, 64)), full((64,)), full((64, 32)), full((32,)),
                  full((32, 1)), full((1,))],
        out_specs=pl.BlockSpec((BM, 1), lambda i: (i, 0)),
        out_shape=jax.ShapeDtypeStruct((B, 1), jnp.float32),
    )(uT, mT, a, c, t, W1, b1, W2, b2, W3, b3)


def kernel(user, movie, actor, country, movie_type,
           user_table, movie_table, actor_table, country_table, type_table,
           W1, b1, W2, b2, W3, b3):
    user = user.astype(jnp.int32)
    actor_t = actor.T
    country_t = country.T
    type_t = movie_type.T
    a, c, t = _sc_pooled(actor_t, country_t, type_t,
                         actor_table, country_table, type_table)
    uflat = _detile(user_table.T, KB_U)
    mflat = _detile(movie_table.T, KB_M)
    u, m = _sc_scalar(user, movie, uflat, mflat)
    y = _mlp(u, m, a, c, t, W1, b1, W2, b2, W3, b3)
    return jnp.squeeze(y, axis=-1)


# speedup vs baseline: 10.2009x; 1.0065x over previous
import functools

import jax
import jax.numpy as jnp
from jax import lax
from jax.experimental import pallas as pl
from jax.experimental.pallas import tpu as pltpu
from jax.experimental.pallas import tpu_sc as plsc

B = 16384
D = 32
NC, NS = 2, 16
NW = NC * NS
BPW = B // NW
N_ACTOR, N_COUNTRY, N_TYPE = 20, 4, 8
UCB = 17
UCW = 1 << UCB
KB_U = 8
KB_M = 1


def _scalar_offsets(idx, kb, dd):
    return ((((dd >> 3) * kb + (idx >> UCB)) << (UCB + 3))
            + ((dd & 7) << UCB) + (idx & (UCW - 1)))


def _sc_scalar_body(user_hbm, movie_hbm, uflat_hbm, mflat_hbm,
                    u_out, m_out,
                    uidx_v, midx_v, eu_v, em_v, du_v, dm_v, sem_s):
    wid = lax.axis_index("s") * NC + lax.axis_index("c")
    base = wid * BPW

    pltpu.sync_copy(user_hbm.at[pl.ds(base, BPW)], uidx_v)
    pltpu.sync_copy(movie_hbm.at[pl.ds(base, BPW)], midx_v)

    def gen_d(dd, carry):
        def gen_i(c, carry2):
            i16 = c * 16
            eu_v[dd, pl.ds(i16, 16)] = _scalar_offsets(
                uidx_v[pl.ds(i16, 16)], KB_U, dd)
            em_v[dd, pl.ds(i16, 16)] = _scalar_offsets(
                midx_v[pl.ds(i16, 16)], KB_M, dd)
            return carry2
        lax.fori_loop(0, BPW // 16, gen_i, 0)
        return carry
    lax.fori_loop(0, D, gen_d, 0)

    def fire(dd, carry):
        pltpu.async_copy(uflat_hbm.at[eu_v.at[dd]], du_v.at[dd], sem_s)
        pltpu.async_copy(mflat_hbm.at[em_v.at[dd]], dm_v.at[dd], sem_s)
        return carry
    lax.fori_loop(0, D, fire, 0)

    def drain(dd, carry):
        pltpu.make_async_copy(uflat_hbm.at[eu_v.at[0]], du_v.at[0], sem_s).wait()
        pltpu.make_async_copy(mflat_hbm.at[em_v.at[0]], dm_v.at[0], sem_s).wait()
        return carry
    lax.fori_loop(0, D, drain, 0)
    pltpu.sync_copy(du_v, u_out.at[:, pl.ds(base, BPW)])
    pltpu.sync_copy(dm_v, m_out.at[:, pl.ds(base, BPW)])


def _sc_scalar(user, movie, uflat, mflat):
    embT = jax.ShapeDtypeStruct((D, B), jnp.float32)
    run = pl.kernel(
        _sc_scalar_body,
        out_type=(embT, embT),
        mesh=plsc.VectorSubcoreMesh(core_axis_name="c", subcore_axis_name="s",
                                    num_cores=NC, num_subcores=NS),
        scratch_types=[
            pltpu.VMEM((BPW,), jnp.int32),
            pltpu.VMEM((BPW,), jnp.int32),
            pltpu.VMEM((D, BPW), jnp.int32),
            pltpu.VMEM((D, BPW), jnp.int32),
            pltpu.VMEM((D, BPW), jnp.float32),
            pltpu.VMEM((D, BPW), jnp.float32),
            pltpu.SemaphoreType.DMA,
        ],
        compiler_params=pltpu.CompilerParams(use_tc_tiling_on_sc=False),
    )
    return run(user, movie, uflat, mflat)


def _sc_pooled_body(actor_hbm, country_hbm, type_hbm,
                    at_hbm, ct_hbm, tt_hbm,
                    a_out, c_out, t_out,
                    aidx_v, cidx_v, tidx_v, acc_a, acc_a2, acc_c, acc_t,
                    sem_a, sem_a2, sem_c, sem_t):
    wid = lax.axis_index("s") * NC + lax.axis_index("c")
    base = wid * BPW

    pltpu.sync_copy(actor_hbm.at[:, pl.ds(base, BPW)], aidx_v)
    pltpu.sync_copy(country_hbm.at[:, pl.ds(base, BPW)], cidx_v)
    pltpu.sync_copy(type_hbm.at[:, pl.ds(base, BPW)], tidx_v)

    def fire_p(table, idx_v, j, acc, sem, add):
        pltpu.async_copy(table.at[idx_v.at[j]], acc, sem, add=add)

    def wait_p(table, idx_v, acc, sem):
        pltpu.make_async_copy(table.at[idx_v.at[0]], acc, sem).wait()

    fire_p(at_hbm, aidx_v, 0, acc_a, sem_a, False)
    fire_p(at_hbm, aidx_v, 1, acc_a2, sem_a2, False)
    fire_p(ct_hbm, cidx_v, 0, acc_c, sem_c, False)
    fire_p(tt_hbm, tidx_v, 0, acc_t, sem_t, False)

    def step4(j, carry):
        wait_p(at_hbm, aidx_v, acc_a, sem_a)
        fire_p(at_hbm, aidx_v, 2 * j, acc_a, sem_a, True)
        wait_p(at_hbm, aidx_v, acc_a2, sem_a2)
        fire_p(at_hbm, aidx_v, 2 * j + 1, acc_a2, sem_a2, True)
        wait_p(ct_hbm, cidx_v, acc_c, sem_c)
        fire_p(ct_hbm, cidx_v, j, acc_c, sem_c, True)
        wait_p(tt_hbm, tidx_v, acc_t, sem_t)
        fire_p(tt_hbm, tidx_v, j, acc_t, sem_t, True)
        return carry
    lax.fori_loop(1, N_COUNTRY, step4, 0)

    def step3(j, carry):
        wait_p(at_hbm, aidx_v, acc_a, sem_a)
        fire_p(at_hbm, aidx_v, 2 * j, acc_a, sem_a, True)
        wait_p(at_hbm, aidx_v, acc_a2, sem_a2)
        fire_p(at_hbm, aidx_v, 2 * j + 1, acc_a2, sem_a2, True)
        wait_p(tt_hbm, tidx_v, acc_t, sem_t)
        fire_p(tt_hbm, tidx_v, j, acc_t, sem_t, True)
        return carry
    lax.fori_loop(N_COUNTRY, N_TYPE, step3, 0)

    def step2(j, carry):
        wait_p(at_hbm, aidx_v, acc_a, sem_a)
        fire_p(at_hbm, aidx_v, 2 * j, acc_a, sem_a, True)
        wait_p(at_hbm, aidx_v, acc_a2, sem_a2)
        fire_p(at_hbm, aidx_v, 2 * j + 1, acc_a2, sem_a2, True)
        return carry
    lax.fori_loop(N_TYPE, N_ACTOR // 2, step2, 0)

    wait_p(ct_hbm, cidx_v, acc_c, sem_c)
    pltpu.sync_copy(acc_c, c_out.at[pl.ds(base, BPW)])
    wait_p(tt_hbm, tidx_v, acc_t, sem_t)
    pltpu.sync_copy(acc_t, t_out.at[pl.ds(base, BPW)])
    wait_p(at_hbm, aidx_v, acc_a, sem_a)
    wait_p(at_hbm, aidx_v, acc_a2, sem_a2)

    def merge(i, carry):
        r = i >> 1
        c16 = (i & 1) * 16
        acc_a[r, pl.ds(c16, 16)] = (acc_a[r, pl.ds(c16, 16)]
                                    + acc_a2[r, pl.ds(c16, 16)])
        return carry
    lax.fori_loop(0, BPW * 2, merge, 0)
    pltpu.sync_copy(acc_a, a_out.at[pl.ds(base, BPW)])


def _sc_pooled(actor_t, country_t, type_t,
               actor_table, country_table, type_table):
    emb = jax.ShapeDtypeStruct((B, D), jnp.float32)
    run = pl.kernel(
        _sc_pooled_body,
        out_type=(emb, emb, emb),
        mesh=plsc.VectorSubcoreMesh(core_axis_name="c", subcore_axis_name="s",
                                    num_cores=NC, num_subcores=NS),
        scratch_types=[
            pltpu.VMEM((N_ACTOR, BPW), jnp.int32),
            pltpu.VMEM((N_COUNTRY, BPW), jnp.int32),
            pltpu.VMEM((N_TYPE, BPW), jnp.int32),
            pltpu.VMEM((BPW, D), jnp.float32),
            pltpu.VMEM((BPW, D), jnp.float32),
            pltpu.VMEM((BPW, D), jnp.float32),
            pltpu.VMEM((BPW, D), jnp.float32),
            pltpu.SemaphoreType.DMA,
            pltpu.SemaphoreType.DMA,
            pltpu.SemaphoreType.DMA,
            pltpu.SemaphoreType.DMA,
        ],
        compiler_params=pltpu.CompilerParams(use_tc_tiling_on_sc=False),
    )
    return run(actor_t, country_t, type_t,
               actor_table, country_table, type_table)


def _detile_body(inp, out):
    out[...] = inp[...].reshape(8 * UCW)


def _detile(tT, kb):
    grid = (D // 8, kb)
    return pl.pallas_call(
        _detile_body,
        grid=grid,
        in_specs=[pl.BlockSpec((8, UCW), lambda d8, k: (d8, k))],
        out_specs=pl.BlockSpec((8 * UCW,), lambda d8, k: (d8 * kb + k,)),
        out_shape=jax.ShapeDtypeStruct((D * kb * UCW,), jnp.float32),
    )(tT)


def _mlp_body(uT, mT, a, c, t, w1, b1, w2, b2, w3, b3, out):
    f32 = jnp.float32
    dn0 = (((0,), (0,)), ((), ()))
    h = (lax.dot_general(uT[...], w1[0:D, :], dn0, preferred_element_type=f32)
         + lax.dot_general(mT[...], w1[D:2 * D, :], dn0, preferred_element_type=f32)
         + jnp.dot(a[...] * (1.0 / N_ACTOR), w1[2 * D:3 * D, :], preferred_element_type=f32)
         + jnp.dot(c[...] * (1.0 / N_COUNTRY), w1[3 * D:4 * D, :], preferred_element_type=f32)
         + jnp.dot(t[...] * (1.0 / N_TYPE), w1[4 * D:5 * D, :], preferred_element_type=f32)
         + b1[...])
    h = jnp.maximum(h, 0.0)
    h2 = jnp.maximum(jnp.dot(h, w2[...], preferred_element_type=f32) + b2[...], 0.0)
    out[...] = jnp.dot(h2, w3[...], preferred_element_type=f32) + b3[...]


def _mlp(uT, mT, a, c, t, W1, b1, W2, b2, W3, b3):
    BM = 4096
    grid = (B // BM,)
    emb_spec = pl.BlockSpec((BM, D), lambda i: (i, 0))
    embT_spec = pl.BlockSpec((D, BM), lambda i: (0, i))
    full = lambda s: pl.BlockSpec(s, lambda i: tuple(0 for _ in s))
    return pl.pallas_call(
        _mlp_body,
        grid=grid,
        in_specs=[embT_spec, embT_spec, emb_spec, emb_spec, emb_spec,
                  full((5 * D, 64)), full((64,)), full((64, 32)), full((32,)),
                  full((32, 1)), full((1,))],
        out_specs=pl.BlockSpec((BM, 1), lambda i: (i, 0)),
        out_shape=jax.ShapeDtypeStruct((B, 1), jnp.float32),
    )(uT, mT, a, c, t, W1, b1, W2, b2, W3, b3)


def kernel(user, movie, actor, country, movie_type,
           user_table, movie_table, actor_table, country_table, type_table,
           W1, b1, W2, b2, W3, b3):
    user = user.astype(jnp.int32)
    actor_t = actor.T
    country_t = country.T
    type_t = movie_type.T
    a, c, t = _sc_pooled(actor_t, country_t, type_t,
                         actor_table, country_table, type_table)
    uflat = _detile(user_table.T, KB_U)
    mflat = _detile(movie_table.T, KB_M)
    u, m = _sc_scalar(user, movie, uflat, mflat)
    y = _mlp(u, m, a, c, t, W1, b1, W2, b2, W3, b3)
    return jnp.squeeze(y, axis=-1)
